# baseline (speedup 1.0000x reference)
import functools

import jax
import jax.numpy as jnp
from jax.experimental import pallas as pl
from jax.experimental.pallas import tpu as pltpu

_K = 5


def _corr_topk_body(x_ref, idx_ref, w_ref, y_ref, *, rows, seq_len):
    rb = pl.program_id(1)
    num_vars = x_ref.shape[1]

    @pl.when(rb == 0)
    def _normalize():
        xb = x_ref[0]
        mean = jnp.mean(xb, axis=-1, keepdims=True)
        d = xb - mean
        var = jnp.sum(d * d, axis=-1, keepdims=True) / (seq_len - 1)
        y_ref[...] = d / (jnp.sqrt(var) + 1e-8)

    y_full = y_ref[...]
    y_rows = y_ref[pl.ds(rb * rows, rows), :]

    corr = jax.lax.dot_general(
        y_rows, y_full,
        dimension_numbers=(((1,), (1,)), ((), ())),
        preferred_element_type=jnp.float32,
    ) * (1.0 / seq_len)

    colsf = jax.lax.broadcasted_iota(
        jnp.int32, (rows, num_vars), 1).astype(jnp.float32)
    rowf = jnp.float32(rb * rows) + jax.lax.broadcasted_iota(
        jnp.int32, (rows, num_vars), 0).astype(jnp.float32)
    neg = jnp.float32(-jnp.inf)
    big = jnp.float32(num_vars)

    work = jnp.where(colsf == rowf, neg, corr)

    vals = []
    idxs = []
    for j in range(_K):
        m = jnp.max(work, axis=1, keepdims=True)
        imf = jnp.min(
            jnp.where(work == m, colsf, big), axis=1, keepdims=True)
        vals.append(m)
        idxs.append(imf)
        if j + 1 < _K:
            work = jnp.where(colsf == imf, neg, work)

    idx_ref[0] = jnp.concatenate(idxs, axis=1).astype(jnp.int32)
    w_ref[0] = jnp.concatenate(vals, axis=1)


def kernel(x):
    batch, num_vars, seq_len = x.shape
    rows = 512
    grid = (batch, num_vars // rows)

    idx, w = pl.pallas_call(
        functools.partial(_corr_topk_body, rows=rows, seq_len=seq_len),
        grid=grid,
        in_specs=[
            pl.BlockSpec((1, num_vars, seq_len), lambda b, rb: (b, 0, 0)),
        ],
        out_specs=[
            pl.BlockSpec((1, rows, _K), lambda b, rb: (b, rb, 0)),
            pl.BlockSpec((1, rows, _K), lambda b, rb: (b, rb, 0)),
        ],
        out_shape=[
            jax.ShapeDtypeStruct((batch, num_vars, _K), jnp.int32),
            jax.ShapeDtypeStruct((batch, num_vars, _K), jnp.float32),
        ],
        scratch_shapes=[pltpu.VMEM((num_vars, seq_len), jnp.float32)],
    )(x)

    offsets = (jnp.arange(batch) * num_vars)[:, None, None]
    src = jnp.broadcast_to(
        jnp.arange(num_vars)[None, :, None], (batch, num_vars, _K)) + offsets
    dst = idx + offsets
    edge_index = jnp.stack(
        [src.reshape(-1), dst.reshape(-1)], axis=0).astype(jnp.int64)
    edge_weight = w.reshape(-1).astype(jnp.float32)
    return edge_index, edge_weight


# speedup vs baseline: 1.0053x; 1.0053x over previous
import functools

import jax
import jax.numpy as jnp
from jax.experimental import pallas as pl
from jax.experimental.pallas import tpu as pltpu

_K = 5


def _corr_topk_body(x_ref, idx_ref, w_ref, y_ref, *, rows, seq_len):
    rb = pl.program_id(1)
    num_vars = x_ref.shape[1]

    @pl.when(rb == 0)
    def _normalize():
        xb = x_ref[0]
        mean = jnp.mean(xb, axis=-1, keepdims=True)
        d = xb - mean
        var = jnp.sum(d * d, axis=-1, keepdims=True) / (seq_len - 1)
        y_ref[...] = d / (jnp.sqrt(var) + 1e-8)

    y_full = y_ref[...]
    y_rows = y_ref[pl.ds(rb * rows, rows), :]

    corr = jax.lax.dot_general(
        y_rows, y_full,
        dimension_numbers=(((1,), (1,)), ((), ())),
        preferred_element_type=jnp.float32,
    ) * (1.0 / seq_len)

    colsf = jax.lax.broadcasted_iota(
        jnp.int32, (rows, num_vars), 1).astype(jnp.float32)
    rowf = jnp.float32(rb * rows) + jax.lax.broadcasted_iota(
        jnp.int32, (rows, num_vars), 0).astype(jnp.float32)
    neg = jnp.float32(-jnp.inf)
    big = jnp.float32(num_vars)

    work = jnp.where(colsf == rowf, neg, corr)

    vals = []
    idxs = []
    for j in range(_K):
        m = jnp.max(work, axis=1, keepdims=True)
        imf = jnp.min(
            jnp.where(work == m, colsf, big), axis=1, keepdims=True)
        vals.append(m)
        idxs.append(imf)
        if j + 1 < _K:
            work = jnp.where(colsf == imf, neg, work)

    idx_ref[0] = jnp.concatenate(idxs, axis=1).astype(jnp.int32)
    w_ref[0] = jnp.concatenate(vals, axis=1)


def kernel(x):
    batch, num_vars, seq_len = x.shape
    rows = 1024
    grid = (batch, num_vars // rows)

    idx, w = pl.pallas_call(
        functools.partial(_corr_topk_body, rows=rows, seq_len=seq_len),
        grid=grid,
        in_specs=[
            pl.BlockSpec((1, num_vars, seq_len), lambda b, rb: (b, 0, 0)),
        ],
        out_specs=[
            pl.BlockSpec((1, rows, _K), lambda b, rb: (b, rb, 0)),
            pl.BlockSpec((1, rows, _K), lambda b, rb: (b, rb, 0)),
        ],
        out_shape=[
            jax.ShapeDtypeStruct((batch, num_vars, _K), jnp.int32),
            jax.ShapeDtypeStruct((batch, num_vars, _K), jnp.float32),
        ],
        scratch_shapes=[pltpu.VMEM((num_vars, seq_len), jnp.float32)],
    )(x)

    offsets = (jnp.arange(batch) * num_vars)[:, None, None]
    src = jnp.broadcast_to(
        jnp.arange(num_vars)[None, :, None], (batch, num_vars, _K)) + offsets
    dst = idx + offsets
    edge_index = jnp.stack(
        [src.reshape(-1), dst.reshape(-1)], axis=0).astype(jnp.int64)
    edge_weight = w.reshape(-1).astype(jnp.float32)
    return edge_index, edge_weight
